# Initial kernel scaffold; baseline (speedup 1.0000x reference)
import functools

import jax
import jax.numpy as jnp
from jax.experimental import pallas as pl
from jax.experimental.pallas import tpu as pltpu

N, D_MODEL, D_FF, C = 4096, 2048, 4096, 2
BLK_N = 512
BLK_F = 512
NF = D_FF // BLK_F
R = N // BLK_N


def _mlp_kernel(x_ref, m_ref,
                w1a1_ref, w2a1_ref, w1b1_ref, w2b1_ref,
                w1a2_ref, w2a2_ref, w1b2_ref, w2b2_ref,
                out1_ref, out2_ref, pred_ref,
                acc1_ref, acc2_ref):
    f = pl.program_id(0)
    i = pl.program_id(1)
    xb = x_ref[...]

    def stream(w1_ref, w2_ref):
        h = jax.nn.gelu(jnp.dot(xb, w1_ref[...],
                                preferred_element_type=jnp.float32))
        return jnp.dot(h, w2_ref[...], preferred_element_type=jnp.float32)

    c1 = stream(w1a1_ref, w2a1_ref) + stream(w1b1_ref, w2b1_ref)
    c2 = stream(w1a2_ref, w2a2_ref) + stream(w1b2_ref, w2b2_ref)

    sl = pl.ds(i * BLK_N, BLK_N)

    @pl.when(f == 0)
    def _():
        acc1_ref[sl, :] = c1
        acc2_ref[sl, :] = c2

    @pl.when(f > 0)
    def _():
        acc1_ref[sl, :] += c1
        acc2_ref[sl, :] += c2

    @pl.when(f == NF - 1)
    def _():
        l1 = acc1_ref[sl, :]
        l2 = acc2_ref[sl, :]
        out1_ref[...] = l1
        mask = m_ref[...] != 0
        out2_ref[...] = jnp.where(mask, l2, 0.0)
        pred_ref[...] = (l1[:, 1:2] > l1[:, 0:1]).astype(jnp.float32)


@jax.jit
def _fused(x, mask_i32, l1_W1a, l1_W2a, l1_W1b, l1_W2b,
           l2_W1a, l2_W2a, l2_W1b, l2_W2b):
    row_blk = lambda f, i: (i, 0)
    w1_blk = lambda f, i: (0, f)
    w2_blk = lambda f, i: (f, 0)
    return pl.pallas_call(
        _mlp_kernel,
        grid=(NF, R),
        in_specs=[
            pl.BlockSpec((BLK_N, D_MODEL), row_blk),
            pl.BlockSpec((BLK_N, 1), row_blk),
            pl.BlockSpec((D_MODEL, BLK_F), w1_blk),
            pl.BlockSpec((BLK_F, C), w2_blk),
            pl.BlockSpec((D_MODEL, BLK_F), w1_blk),
            pl.BlockSpec((BLK_F, C), w2_blk),
            pl.BlockSpec((D_MODEL, BLK_F), w1_blk),
            pl.BlockSpec((BLK_F, C), w2_blk),
            pl.BlockSpec((D_MODEL, BLK_F), w1_blk),
            pl.BlockSpec((BLK_F, C), w2_blk),
        ],
        out_specs=[
            pl.BlockSpec((BLK_N, C), row_blk),
            pl.BlockSpec((BLK_N, C), row_blk),
            pl.BlockSpec((BLK_N, 1), row_blk),
        ],
        out_shape=[
            jax.ShapeDtypeStruct((N, C), jnp.float32),
            jax.ShapeDtypeStruct((N, C), jnp.float32),
            jax.ShapeDtypeStruct((N, 1), jnp.float32),
        ],
        scratch_shapes=[
            pltpu.VMEM((N, C), jnp.float32),
            pltpu.VMEM((N, C), jnp.float32),
        ],
    )(x, mask_i32, l1_W1a, l1_W2a, l1_W1b, l1_W2b,
      l2_W1a, l2_W2a, l2_W1b, l2_W2b)


def kernel(x, mask, l1_W1a, l1_W2a, l1_W1b, l1_W2b,
           l2_W1a, l2_W2a, l2_W1b, l2_W2b):
    mask_i32 = mask.astype(jnp.int32).reshape(N, 1)
    out1, out2, predf = _fused(x, mask_i32, l1_W1a, l1_W2a, l1_W1b, l1_W2b,
                               l2_W1a, l2_W2a, l2_W1b, l2_W2b)
    pred = predf.reshape(N).astype(jnp.bool_)
    return (out1, out2, pred)


# speedup vs baseline: 1.0484x; 1.0484x over previous
import jax
import jax.numpy as jnp
from jax.experimental import pallas as pl
from jax.experimental.pallas import tpu as pltpu

N, D_MODEL, D_FF, C = 4096, 2048, 4096, 2
BLK_N = 1024
BLK_F = 512
NF = D_FF // BLK_F
R = N // BLK_N


def _mlp_kernel(x_ref, m_ref,
                w1a1_ref, w2a1_ref, w1b1_ref, w2b1_ref,
                w1a2_ref, w2a2_ref, w1b2_ref, w2b2_ref,
                out1_ref, out2_ref, pred_ref):
    f = pl.program_id(1)
    xb = x_ref[...]

    def stream(w1_ref, w2_ref):
        h = jax.nn.gelu(jnp.dot(xb, w1_ref[...],
                                preferred_element_type=jnp.float32))
        return jnp.dot(h, w2_ref[...], preferred_element_type=jnp.float32)

    c1 = stream(w1a1_ref, w2a1_ref) + stream(w1b1_ref, w2b1_ref)
    c2 = stream(w1a2_ref, w2a2_ref) + stream(w1b2_ref, w2b2_ref)

    @pl.when(f == 0)
    def _():
        out1_ref[...] = c1
        out2_ref[...] = c2

    @pl.when(f > 0)
    def _():
        out1_ref[...] += c1
        out2_ref[...] += c2

    @pl.when(f == NF - 1)
    def _():
        l1 = out1_ref[...]
        mask = m_ref[...] != 0
        out2_ref[...] = jnp.where(mask, out2_ref[...], 0.0)
        pred_ref[...] = (l1[:, 1:2] > l1[:, 0:1]).astype(jnp.float32)


@jax.jit
def _fused(x, mask_i32, l1_W1a, l1_W2a, l1_W1b, l1_W2b,
           l2_W1a, l2_W2a, l2_W1b, l2_W2b):
    row_blk = lambda i, f: (i, 0)
    w1_blk = lambda i, f: (0, f)
    w2_blk = lambda i, f: (f, 0)
    return pl.pallas_call(
        _mlp_kernel,
        grid=(R, NF),
        in_specs=[
            pl.BlockSpec((BLK_N, D_MODEL), row_blk),
            pl.BlockSpec((BLK_N, 1), row_blk),
            pl.BlockSpec((D_MODEL, BLK_F), w1_blk),
            pl.BlockSpec((BLK_F, C), w2_blk),
            pl.BlockSpec((D_MODEL, BLK_F), w1_blk),
            pl.BlockSpec((BLK_F, C), w2_blk),
            pl.BlockSpec((D_MODEL, BLK_F), w1_blk),
            pl.BlockSpec((BLK_F, C), w2_blk),
            pl.BlockSpec((D_MODEL, BLK_F), w1_blk),
            pl.BlockSpec((BLK_F, C), w2_blk),
        ],
        out_specs=[
            pl.BlockSpec((BLK_N, C), row_blk),
            pl.BlockSpec((BLK_N, C), row_blk),
            pl.BlockSpec((BLK_N, 1), row_blk),
        ],
        out_shape=[
            jax.ShapeDtypeStruct((N, C), jnp.float32),
            jax.ShapeDtypeStruct((N, C), jnp.float32),
            jax.ShapeDtypeStruct((N, 1), jnp.float32),
        ],
    )(x, mask_i32, l1_W1a, l1_W2a, l1_W1b, l1_W2b,
      l2_W1a, l2_W2a, l2_W1b, l2_W2b)


def kernel(x, mask, l1_W1a, l1_W2a, l1_W1b, l1_W2b,
           l2_W1a, l2_W2a, l2_W1b, l2_W2b):
    mask_i32 = mask.astype(jnp.int32).reshape(N, 1)
    out1, out2, predf = _fused(x, mask_i32, l1_W1a, l1_W2a, l1_W1b, l1_W2b,
                               l2_W1a, l2_W2a, l2_W1b, l2_W2b)
    pred = predf.reshape(N).astype(jnp.bool_)
    return (out1, out2, pred)
